# baseline (speedup 1.0000x reference)
import numpy as np

import jax
import jax.numpy as jnp
from jax import lax
from jax.experimental import pallas as pl
from jax.experimental.pallas import tpu as pltpu
from jax.experimental.pallas import tpu_sc as plsc

VOCAB = 100000
SEQ_LEN = 2048
DIM = 768
BATCH = 4
N = 10000

NUM_CORES = 2
NUM_SUBCORES = 16
NW = NUM_CORES * NUM_SUBCORES
SCHUNK = 32
NCHUNK = SEQ_LEN // SCHUNK
CHUNKS_PER_W = NCHUNK // NW
LANES = 16
PAIRS_PER_ROW = DIM // (2 * LANES)
NSTEP = CHUNKS_PER_W * BATCH
NBUF = 3


def _positional_table_packed():
    positions = np.arange(0, SEQ_LEN, dtype=np.float32)[:, None]
    den_even = np.power(float(N), 2.0 * np.arange(0, DIM, 2, dtype=np.float32) / DIM)
    den_odd = np.power(float(N), 2.0 * np.arange(1, DIM, 2, dtype=np.float32) / DIM)
    emb = np.zeros((SEQ_LEN, DIM), dtype=np.float32)
    emb[:, 0::2] = np.sin(positions / den_even)
    emb[:, 1::2] = np.cos(positions / den_odd)
    bits = emb.view(np.uint32)
    bf = ((bits + 0x7FFF + ((bits >> 16) & 1)) >> 16).astype(np.uint32)
    bf3 = bf.reshape(SEQ_LEN, DIM // 32, 2, LANES)
    packed = bf3[:, :, 0, :] | (bf3[:, :, 1, :] << 16)
    return jnp.asarray(packed.reshape(SEQ_LEN, DIM // 2), dtype=jnp.uint32)


def _step_addr(wid, i):
    k, b = divmod(i, BATCH)
    s0 = (wid * CHUNKS_PER_W + k) * SCHUNK
    return s0, b * SEQ_LEN + s0


def _sc_body(table_hbm, x_hbm, pos_hbm, out_hbm,
             idx_all, pos0, pos1, rows0, rows1, rows2,
             isem, ppsem, gsem0, gsem1, gsem2, osem0, osem1, osem2):
    wid = lax.axis_index("s") * NUM_CORES + lax.axis_index("c")
    pos_v = (pos0, pos1)
    rows_v = (rows0, rows1, rows2)
    gsem = (gsem0, gsem1, gsem2)
    osem = (osem0, osem1, osem2)

    cps = []
    for i in range(NSTEP):
        k, b = divmod(i, BATCH)
        s0 = (wid * CHUNKS_PER_W + k) * SCHUNK
        cps.append(pltpu.async_copy(
            x_hbm.at[pl.ds(b * SEQ_LEN + s0, SCHUNK)], idx_all.at[i], isem))
    for k in range(CHUNKS_PER_W):
        s0 = (wid * CHUNKS_PER_W + k) * SCHUNK
        cps.append(pltpu.async_copy(
            pos_hbm.at[pl.ds(s0, SCHUNK)], pos_v[k], ppsem))
    for cp in cps:
        cp.wait()

    def fire_gather(i):
        slot = i % NBUF
        return pltpu.async_copy(
            table_hbm.at[idx_all.at[i]], rows_v[slot], gsem[slot])

    gathers = [None] * NBUF
    out_writes = [None] * NBUF
    gathers[0] = fire_gather(0)
    gathers[1] = fire_gather(1)

    hi_mask = jnp.full((LANES,), 0xFFFF0000, jnp.uint32)
    shift16 = jnp.full((LANES,), 16, jnp.uint32)

    for i in range(NSTEP):
        slot = i % NBUF
        k = i // BATCH
        if i + 2 < NSTEP:
            nslot = (i + 2) % NBUF
            if out_writes[nslot] is not None:
                out_writes[nslot].wait()
                out_writes[nslot] = None
            gathers[nslot] = fire_gather(i + 2)
        gathers[slot].wait()
        rv = rows_v[slot]
        pv = pos_v[k]

        def row_add(r, _, rv=rv, pv=pv):
            for p in range(PAIRS_PER_ROW):
                w = pv[r, pl.ds(p * LANES, LANES)]
                lo = lax.bitcast_convert_type(w << shift16, jnp.float32)
                hi = lax.bitcast_convert_type(w & hi_mask, jnp.float32)
                plsc.addupdate(rv.at[r, pl.ds(p * 2 * LANES, LANES)], lo)
                plsc.addupdate(rv.at[r, pl.ds(p * 2 * LANES + LANES, LANES)], hi)
            return 0

        lax.fori_loop(0, SCHUNK, row_add, 0)
        _, base = _step_addr(wid, i)
        out_writes[slot] = pltpu.async_copy(
            rv, out_hbm.at[pl.ds(base, SCHUNK)], osem[slot])
    for w in out_writes:
        if w is not None:
            w.wait()


def kernel(x, token_table):
    pos = _positional_table_packed()
    x32 = x.reshape(-1).astype(jnp.int32)
    mesh = plsc.VectorSubcoreMesh(core_axis_name="c", subcore_axis_name="s")
    out = pl.kernel(
        _sc_body,
        mesh=mesh,
        out_type=jax.ShapeDtypeStruct((BATCH * SEQ_LEN, DIM), jnp.float32),
        scratch_types=[
            pltpu.VMEM((NSTEP, SCHUNK), jnp.int32),
            pltpu.VMEM((SCHUNK, DIM // 2), jnp.uint32),
            pltpu.VMEM((SCHUNK, DIM // 2), jnp.uint32),
            pltpu.VMEM((SCHUNK, DIM), jnp.float32),
            pltpu.VMEM((SCHUNK, DIM), jnp.float32),
            pltpu.VMEM((SCHUNK, DIM), jnp.float32),
        ] + [pltpu.SemaphoreType.DMA] * 8,
    )(token_table, x32, pos)
    return out.reshape(BATCH, SEQ_LEN, DIM)


# speedup vs baseline: 1.1844x; 1.1844x over previous
import numpy as np

import jax
import jax.numpy as jnp
from jax import lax
from jax.experimental import pallas as pl
from jax.experimental.pallas import tpu as pltpu
from jax.experimental.pallas import tpu_sc as plsc

VOCAB = 100000
SEQ_LEN = 2048
DIM = 768
BATCH = 4
N = 10000

NUM_CORES = 2
NUM_SUBCORES = 16
NW = NUM_CORES * NUM_SUBCORES
SCHUNK = 32
NCHUNK = SEQ_LEN // SCHUNK
CHUNKS_PER_W = NCHUNK // NW
LANES = 16
VECS_PER_ROW = DIM // LANES
PAIRS_PER_ROW = DIM // (2 * LANES)
NSTEP = CHUNKS_PER_W * BATCH
NBUF = 3


def _positional_table_packed():
    positions = np.arange(0, SEQ_LEN, dtype=np.float32)[:, None]
    den_even = np.power(float(N), 2.0 * np.arange(0, DIM, 2, dtype=np.float32) / DIM)
    den_odd = np.power(float(N), 2.0 * np.arange(1, DIM, 2, dtype=np.float32) / DIM)
    emb = np.zeros((SEQ_LEN, DIM), dtype=np.float32)
    emb[:, 0::2] = np.sin(positions / den_even)
    emb[:, 1::2] = np.cos(positions / den_odd)
    bits = emb.view(np.uint32)
    bf = ((bits + 0x7FFF + ((bits >> 16) & 1)) >> 16).astype(np.uint32)
    bf3 = bf.reshape(SEQ_LEN, DIM // 32, 2, LANES)
    packed = bf3[:, :, 0, :] | (bf3[:, :, 1, :] << 16)
    return jnp.asarray(packed.reshape(SEQ_LEN, DIM // 2), dtype=jnp.uint32)


def _step_addr(wid, i):
    k, b = divmod(i, BATCH)
    s0 = (wid * CHUNKS_PER_W + k) * SCHUNK
    return s0, b * SEQ_LEN + s0


def _sc_body(table_hbm, x_hbm, pos_hbm, out_hbm,
             idx_all, posp, pos_f, rows0, rows1, rows2,
             isem, ppsem, gsem0, gsem1, gsem2, osem0, osem1, osem2):
    wid = lax.axis_index("s") * NUM_CORES + lax.axis_index("c")
    rows_v = (rows0, rows1, rows2)
    gsem = (gsem0, gsem1, gsem2)
    osem = (osem0, osem1, osem2)

    cps = []
    for i in range(NSTEP):
        s0, base = _step_addr(wid, i)
        cps.append(pltpu.async_copy(
            x_hbm.at[pl.ds(base, SCHUNK)], idx_all.at[i], isem))
    for k in range(CHUNKS_PER_W):
        s0 = (wid * CHUNKS_PER_W + k) * SCHUNK
        cps.append(pltpu.async_copy(
            pos_hbm.at[pl.ds(s0, SCHUNK)], posp.at[k], ppsem))
    for cp in cps:
        cp.wait()

    def fire_gather(i):
        slot = i % NBUF
        return pltpu.async_copy(
            table_hbm.at[idx_all.at[i]], rows_v[slot], gsem[slot])

    gathers = [None] * NBUF
    out_writes = [None] * NBUF
    gathers[0] = fire_gather(0)
    gathers[1] = fire_gather(1)

    hi_mask = jnp.full((LANES,), 0xFFFF0000, jnp.uint32)
    shift16 = jnp.full((LANES,), 16, jnp.uint32)

    def unpack_chunk(k):
        def unpack_row(r, _, k=k):
            for p in range(PAIRS_PER_ROW):
                w = posp[k, r, pl.ds(p * LANES, LANES)]
                pos_f[r, pl.ds(p * 2 * LANES, LANES)] = (
                    lax.bitcast_convert_type(w << shift16, jnp.float32))
                pos_f[r, pl.ds(p * 2 * LANES + LANES, LANES)] = (
                    lax.bitcast_convert_type(w & hi_mask, jnp.float32))
            return 0
        lax.fori_loop(0, SCHUNK, unpack_row, 0)

    unpack_chunk(0)

    for i in range(NSTEP):
        slot = i % NBUF
        k = i // BATCH
        if i + 2 < NSTEP:
            nslot = (i + 2) % NBUF
            if out_writes[nslot] is not None:
                out_writes[nslot].wait()
                out_writes[nslot] = None
            gathers[nslot] = fire_gather(i + 2)
        gathers[slot].wait()
        rv = rows_v[slot]

        def row_add(r, _, rv=rv):
            for j in range(VECS_PER_ROW):
                sl = pl.ds(j * LANES, LANES)
                plsc.addupdate(rv.at[r, sl], pos_f[r, sl])
            return 0

        lax.fori_loop(0, SCHUNK, row_add, 0)
        _, base = _step_addr(wid, i)
        out_writes[slot] = pltpu.async_copy(
            rv, out_hbm.at[pl.ds(base, SCHUNK)], osem[slot])
        if i == BATCH - 1 and CHUNKS_PER_W > 1:
            unpack_chunk(1)
    for w in out_writes:
        if w is not None:
            w.wait()


def kernel(x, token_table):
    pos = _positional_table_packed()
    x_flat = x.reshape(-1).astype(jnp.int32)
    mesh = plsc.VectorSubcoreMesh(core_axis_name="c", subcore_axis_name="s")
    out = pl.kernel(
        _sc_body,
        mesh=mesh,
        out_type=jax.ShapeDtypeStruct((BATCH * SEQ_LEN, DIM), jnp.float32),
        scratch_types=[
            pltpu.VMEM((NSTEP, SCHUNK), jnp.int32),
            pltpu.VMEM((CHUNKS_PER_W, SCHUNK, DIM // 2), jnp.uint32),
            pltpu.VMEM((SCHUNK, DIM), jnp.float32),
            pltpu.VMEM((SCHUNK, DIM), jnp.float32),
            pltpu.VMEM((SCHUNK, DIM), jnp.float32),
            pltpu.VMEM((SCHUNK, DIM), jnp.float32),
        ] + [pltpu.SemaphoreType.DMA] * 8,
    )(token_table, x_flat, pos)
    return out.reshape(BATCH, SEQ_LEN, DIM)


# speedup vs baseline: 1.2174x; 1.0279x over previous
import numpy as np

import jax
import jax.numpy as jnp
from jax import lax
from jax.experimental import pallas as pl
from jax.experimental.pallas import tpu as pltpu
from jax.experimental.pallas import tpu_sc as plsc

VOCAB = 100000
SEQ_LEN = 2048
DIM = 768
BATCH = 4
N = 10000

NUM_CORES = 2
NUM_SUBCORES = 16
NW = NUM_CORES * NUM_SUBCORES
SCHUNK = 32
NCHUNK = SEQ_LEN // SCHUNK
CHUNKS_PER_W = NCHUNK // NW
LANES = 16
VECS_PER_ROW = DIM // LANES
NSTEP = CHUNKS_PER_W * BATCH
NBUF = 3


def _positional_table():
    positions = np.arange(0, SEQ_LEN, dtype=np.float32)[:, None]
    den_even = np.power(float(N), 2.0 * np.arange(0, DIM, 2, dtype=np.float32) / DIM)
    den_odd = np.power(float(N), 2.0 * np.arange(1, DIM, 2, dtype=np.float32) / DIM)
    emb = np.zeros((SEQ_LEN, DIM), dtype=np.float32)
    emb[:, 0::2] = np.sin(positions / den_even)
    emb[:, 1::2] = np.cos(positions / den_odd)
    return jnp.asarray(emb)


def _step_addr(wid, i):
    k, b = divmod(i, BATCH)
    s0 = (wid * CHUNKS_PER_W + k) * SCHUNK
    return s0, b * SEQ_LEN + s0


def _sc_body(table_hbm, x_hbm, pos_hbm, out_hbm,
             idx_all, pos0, pos1, rows0, rows1, rows2,
             isem, ppsem, gsem0, gsem1, gsem2, osem0, osem1, osem2):
    wid = lax.axis_index("s") * NUM_CORES + lax.axis_index("c")
    pos_v = (pos0, pos1)
    rows_v = (rows0, rows1, rows2)
    gsem = (gsem0, gsem1, gsem2)
    osem = (osem0, osem1, osem2)

    cps = []
    for i in range(NSTEP):
        k, b = divmod(i, BATCH)
        s0 = (wid * CHUNKS_PER_W + k) * SCHUNK
        cps.append(pltpu.async_copy(
            x_hbm.at[b, pl.ds(s0, SCHUNK)], idx_all.at[i], isem))
    for k in range(CHUNKS_PER_W):
        s0 = (wid * CHUNKS_PER_W + k) * SCHUNK
        cps.append(pltpu.async_copy(
            pos_hbm.at[pl.ds(s0, SCHUNK)], pos_v[k], ppsem))
    for cp in cps:
        cp.wait()

    def fire_gather(i):
        slot = i % NBUF
        return pltpu.async_copy(
            table_hbm.at[idx_all.at[i]], rows_v[slot], gsem[slot])

    gathers = [None] * NBUF
    out_writes = [None] * NBUF
    gathers[0] = fire_gather(0)
    gathers[1] = fire_gather(1)

    for i in range(NSTEP):
        slot = i % NBUF
        k = i // BATCH
        if i + 2 < NSTEP:
            nslot = (i + 2) % NBUF
            if out_writes[nslot] is not None:
                out_writes[nslot].wait()
                out_writes[nslot] = None
            gathers[nslot] = fire_gather(i + 2)
        gathers[slot].wait()
        rv = rows_v[slot]
        pv = pos_v[k]

        def row_add(r, _, rv=rv, pv=pv):
            for j in range(VECS_PER_ROW):
                sl = pl.ds(j * LANES, LANES)
                plsc.addupdate(rv.at[r, sl], pv[r, sl])
            return 0

        lax.fori_loop(0, SCHUNK, row_add, 0)
        _, base = _step_addr(wid, i)
        out_writes[slot] = pltpu.async_copy(
            rv, out_hbm.at[pl.ds(base, SCHUNK)], osem[slot])
    for w in out_writes:
        if w is not None:
            w.wait()


def kernel(x, token_table):
    pos = _positional_table()
    x_flat = x.astype(jnp.int32)
    mesh = plsc.VectorSubcoreMesh(core_axis_name="c", subcore_axis_name="s")
    out = pl.kernel(
        _sc_body,
        mesh=mesh,
        out_type=jax.ShapeDtypeStruct((BATCH * SEQ_LEN, DIM), jnp.float32),
        scratch_types=[
            pltpu.VMEM((NSTEP, SCHUNK), jnp.int32),
            pltpu.VMEM((SCHUNK, DIM), jnp.float32),
            pltpu.VMEM((SCHUNK, DIM), jnp.float32),
            pltpu.VMEM((SCHUNK, DIM), jnp.float32),
            pltpu.VMEM((SCHUNK, DIM), jnp.float32),
            pltpu.VMEM((SCHUNK, DIM), jnp.float32),
        ] + [pltpu.SemaphoreType.DMA] * 8,
    )(token_table, x_flat, pos)
    return out.reshape(BATCH, SEQ_LEN, DIM)


# speedup vs baseline: 1.3206x; 1.0847x over previous
import numpy as np

import jax
import jax.numpy as jnp
from jax import lax
from jax.experimental import pallas as pl
from jax.experimental.pallas import tpu as pltpu
from jax.experimental.pallas import tpu_sc as plsc

VOCAB = 100000
SEQ_LEN = 2048
DIM = 768
BATCH = 4
N = 10000

NUM_CORES = 2
NUM_SUBCORES = 16
NW = NUM_CORES * NUM_SUBCORES
SCHUNK = 16
NCHUNK = SEQ_LEN // SCHUNK
CHUNKS_PER_W = NCHUNK // NW
LANES = 16
VECS_PER_ROW = DIM // LANES


def _positional_table():
    positions = np.arange(0, SEQ_LEN, dtype=np.float32)[:, None]
    den_even = np.power(float(N), 2.0 * np.arange(0, DIM, 2, dtype=np.float32) / DIM)
    den_odd = np.power(float(N), 2.0 * np.arange(1, DIM, 2, dtype=np.float32) / DIM)
    emb = np.zeros((SEQ_LEN, DIM), dtype=np.float32)
    emb[:, 0::2] = np.sin(positions / den_even)
    emb[:, 1::2] = np.cos(positions / den_odd)
    return jnp.asarray(emb)


def _sc_body(table_hbm, x_hbm, pos_hbm, out_hbm,
             idx_all, pos0, pos1,
             r00, r10, r20, r30, r01, r11, r21, r31,
             isem, ppsem0, ppsem1,
             g00, g10, g20, g30, g01, g11, g21, g31,
             o00, o10, o20, o30, o01, o11, o21, o31):
    wid = lax.axis_index("s") * NUM_CORES + lax.axis_index("c")
    pos_v = (pos0, pos1)
    ppsem = (ppsem0, ppsem1)
    rows = ((r00, r10, r20, r30), (r01, r11, r21, r31))
    gsem = ((g00, g10, g20, g30), (g01, g11, g21, g31))
    osem = ((o00, o10, o20, o30), (o01, o11, o21, o31))

    def chunk_s0(k):
        return (wid * CHUNKS_PER_W + k) * SCHUNK

    cps = []
    for k in range(CHUNKS_PER_W):
        s0 = chunk_s0(k)
        for b in range(BATCH):
            cps.append(pltpu.async_copy(
                x_hbm.at[b, pl.ds(s0, SCHUNK)], idx_all.at[k * BATCH + b], isem))
    pos_cp = [None, None]
    for k in range(2):
        pos_cp[k] = pltpu.async_copy(
            pos_hbm.at[pl.ds(chunk_s0(k), SCHUNK)], pos_v[k], ppsem[k])
    for cp in cps:
        cp.wait()

    def fire_gathers(k):
        p = k % 2
        return [pltpu.async_copy(
            table_hbm.at[idx_all.at[k * BATCH + b]], rows[p][b], gsem[p][b])
            for b in range(BATCH)]

    gathers = [None, None]
    out_writes = [None, None]
    gathers[0] = fire_gathers(0)
    gathers[1] = fire_gathers(1)

    for k in range(CHUNKS_PER_W):
        p = k % 2
        for cp in gathers[p]:
            cp.wait()
        pos_cp[p].wait()
        rv = rows[p]
        pv = pos_v[p]

        def row_add(r, _, rv=rv, pv=pv):
            for j in range(VECS_PER_ROW):
                sl = pl.ds(j * LANES, LANES)
                v = pv[r, sl]
                for b in range(BATCH):
                    plsc.addupdate(rv[b].at[r, sl], v)
            return 0

        lax.fori_loop(0, SCHUNK, row_add, 0)

        if k + 2 < CHUNKS_PER_W:
            pos_cp[p] = pltpu.async_copy(
                pos_hbm.at[pl.ds(chunk_s0(k + 2), SCHUNK)], pos_v[p], ppsem[p])
        s0 = chunk_s0(k)
        out_writes[p] = [pltpu.async_copy(
            rv[b], out_hbm.at[pl.ds(b * SEQ_LEN + s0, SCHUNK)], osem[p][b])
            for b in range(BATCH)]
        if k + 2 < CHUNKS_PER_W:
            for cp in out_writes[p]:
                cp.wait()
            out_writes[p] = None
            gathers[p] = fire_gathers(k + 2)
    for ow in out_writes:
        if ow is not None:
            for cp in ow:
                cp.wait()


def kernel(x, token_table):
    pos = _positional_table()
    x32 = x.astype(jnp.int32)
    mesh = plsc.VectorSubcoreMesh(core_axis_name="c", subcore_axis_name="s")
    out = pl.kernel(
        _sc_body,
        mesh=mesh,
        out_type=jax.ShapeDtypeStruct((BATCH * SEQ_LEN, DIM), jnp.float32),
        scratch_types=[
            pltpu.VMEM((CHUNKS_PER_W * BATCH, SCHUNK), jnp.int32),
            pltpu.VMEM((SCHUNK, DIM), jnp.float32),
            pltpu.VMEM((SCHUNK, DIM), jnp.float32),
        ] + [pltpu.VMEM((SCHUNK, DIM), jnp.float32)] * 8
          + [pltpu.SemaphoreType.DMA] * 19,
    )(token_table, x32, pos)
    return out.reshape(BATCH, SEQ_LEN, DIM)
